# baseline (speedup 1.0000x reference)
import functools

import jax
import jax.numpy as jnp
from jax import lax
from jax.experimental import pallas as pl
from jax.experimental.pallas import tpu as pltpu
from jax.experimental.pallas import tpu_sc as plsc

_NLANE = 16
_NCORE = 2
_NSUB = 16

_KB = 2048
_NBLK = 62
_NSTREAM = 8


def _tc_pack_half(tt, j):
    dim, vocab = tt.shape
    hd = dim // 2
    last = pl.cdiv(vocab, _KB) - 1

    def body(*refs):
        o_ref = refs[-1]

        def one(q_ref):
            t = q_ref[...]
            a = lax.slice(t, (0, 0), (hd // 2, _KB))
            b = lax.slice(t, (hd // 2, 0), (hd, _KB))
            au = lax.bitcast_convert_type(
                a.astype(jnp.bfloat16), jnp.uint16).astype(jnp.uint32)
            bu = lax.bitcast_convert_type(
                b.astype(jnp.bfloat16), jnp.uint16).astype(jnp.uint32)
            w = lax.bitcast_convert_type(au | (bu << 16), jnp.float32)
            return jnp.transpose(w)

        o_ref[...] = jnp.concatenate([one(r) for r in refs[:-1]], axis=1)

    def spec(q):
        return pl.BlockSpec(
            (hd, _KB),
            lambda i, q=q: (j, jnp.minimum(i + q * _NBLK, last)))

    return pl.pallas_call(
        body,
        grid=(_NBLK,),
        in_specs=[spec(q) for q in range(_NSTREAM)],
        out_specs=pl.BlockSpec((_KB, 2 * dim), lambda i: (i, 0)),
        out_shape=jax.ShapeDtypeStruct((_NBLK * _KB, 2 * dim), jnp.float32),
    )(*([tt] * _NSTREAM))


def _sc_pool_half(xp, table_lin, seq, hd):
    batch, _ = xp.shape
    nw = _NCORE * _NSUB
    assert batch % nw == 0
    bpw = batch // nw
    assert bpw % 4 == 0
    wdim = hd // 2

    mesh = plsc.VectorSubcoreMesh(core_axis_name="c", subcore_axis_name="s")

    @functools.partial(
        pl.kernel,
        mesh=mesh,
        compiler_params=pltpu.CompilerParams(
            use_tc_tiling_on_sc=False, needs_layout_passes=False),
        out_type=jax.ShapeDtypeStruct((batch, hd), jnp.float32),
        scratch_types=[
            pltpu.VMEM((bpw, seq), jnp.int32),
            pltpu.VMEM((4, seq, wdim), jnp.float32),
            pltpu.VMEM((bpw, hd), jnp.float32),
            pltpu.SemaphoreType.DMA,
            pltpu.SemaphoreType.DMA,
            pltpu.SemaphoreType.DMA,
            pltpu.SemaphoreType.DMA,
        ],
    )
    def k(x_hbm, table_hbm, out_hbm, idx_v, rows_v, pooled_v,
          sem0, sem1, sem2, sem3):
        wid = lax.axis_index("s") * _NCORE + lax.axis_index("c")
        base = wid * bpw
        pltpu.sync_copy(x_hbm.at[pl.ds(base, bpw)], idx_v)

        sems = (sem0, sem1, sem2, sem3)

        def gather(r, buf):
            return pltpu.make_async_copy(
                table_hbm.at[idx_v.at[r]], rows_v.at[buf], sems[buf])

        def accum(r, buf):
            acc = plsc.bitcast(rows_v[buf, 0, pl.ds(0, _NLANE)], jnp.bfloat16)
            for s in range(1, seq):
                acc = acc + plsc.bitcast(
                    rows_v[buf, s, pl.ds(0, _NLANE)], jnp.bfloat16)
            ea, eb = plsc.unpack(acc, format=plsc.PackFormat.INTERLEAVED)
            pooled_v[r, pl.ds(0, _NLANE)] = ea * (1.0 / seq)
            pooled_v[r, pl.ds(_NLANE, _NLANE)] = eb * (1.0 / seq)

        gather(0, 0).start()
        gather(1, 1).start()
        gather(2, 2).start()

        def outer(g, carry):
            r0 = 4 * g
            for bf in range(4):
                r = r0 + bf

                @pl.when(r + 3 < bpw)
                def _():
                    gather(r + 3, (bf + 3) % 4).start()

                gather(r, bf).wait()
                accum(r, bf)
            return carry

        lax.fori_loop(0, bpw // 4, outer, 0)
        pltpu.sync_copy(pooled_v, out_hbm.at[pl.ds(base, bpw)])

    return k(xp, table_lin)


def _tc_head(plo, phi, wt, b2):
    batch, hd = plo.shape
    _, seq = wt.shape
    bb = 2048
    assert batch % bb == 0

    def body(a_ref, c_ref, w_ref, b_ref, o_ref):
        p = jnp.concatenate([a_ref[...], c_ref[...]], axis=1)
        logits = jnp.dot(p, w_ref[...],
                         preferred_element_type=jnp.float32) + b_ref[...]
        o_ref[...] = jax.nn.sigmoid(logits)

    return pl.pallas_call(
        body,
        grid=(batch // bb,),
        in_specs=[
            pl.BlockSpec((bb, hd), lambda i: (i, 0)),
            pl.BlockSpec((bb, hd), lambda i: (i, 0)),
            pl.BlockSpec((2 * hd, seq), lambda i: (0, 0)),
            pl.BlockSpec((1, seq), lambda i: (0, 0)),
        ],
        out_specs=pl.BlockSpec((bb, seq), lambda i: (i, 0)),
        out_shape=jax.ShapeDtypeStruct((batch, seq), jnp.float32),
    )(plo, phi, wt, b2)


def kernel(x, table, W, b):
    vocab, dim = table.shape
    seq = x.shape[1]
    tt = jnp.swapaxes(table, 0, 1)
    ee = _NBLK * _KB
    xp = 8 * lax.rem(x, ee) + lax.div(x, ee)
    packed_lo = _tc_pack_half(tt, 0)
    pooled_lo = _sc_pool_half(xp, packed_lo.reshape(8 * ee, dim // 4),
                              seq, dim // 2)
    packed_hi = _tc_pack_half(tt, 1)
    pooled_hi = _sc_pool_half(xp, packed_hi.reshape(8 * ee, dim // 4),
                              seq, dim // 2)
    return _tc_head(pooled_lo, pooled_hi, W.T, b.reshape(1, -1))


# speedup vs baseline: 1.6934x; 1.6934x over previous
import functools

import jax
import jax.numpy as jnp
import numpy as np
from jax import lax
from jax.experimental import pallas as pl
from jax.experimental.pallas import tpu as pltpu
from jax.experimental.pallas import tpu_sc as plsc

_NLANE = 16
_NCORE = 2
_NSUB = 16

_KB = 4096
_NBLK = 62


def _tc_pack(tt):
    dim, vocab = tt.shape
    last = pl.cdiv(vocab, _KB) - 1

    def body(q0_ref, q1_ref, q2_ref, q3_ref, o_ref):
        def one(q_ref):
            t = q_ref[...]
            a = lax.slice(t, (0, 0), (dim // 2, _KB))
            b = lax.slice(t, (dim // 2, 0), (dim, _KB))
            au = lax.bitcast_convert_type(
                a.astype(jnp.bfloat16), jnp.uint16).astype(jnp.uint32)
            bu = lax.bitcast_convert_type(
                b.astype(jnp.bfloat16), jnp.uint16).astype(jnp.uint32)
            w = lax.bitcast_convert_type(au | (bu << 16), jnp.float32)
            return jnp.transpose(w)

        o_ref[...] = jnp.concatenate(
            [one(q0_ref), one(q1_ref), one(q2_ref), one(q3_ref)], axis=1)

    def spec(q):
        return pl.BlockSpec(
            (dim, _KB), lambda i, q=q: (0, jnp.minimum(i + q * _NBLK, last)))

    return pl.pallas_call(
        body,
        grid=(_NBLK,),
        in_specs=[spec(0), spec(1), spec(2), spec(3)],
        out_specs=pl.BlockSpec((_KB, 2 * dim), lambda i: (i, 0)),
        out_shape=jax.ShapeDtypeStruct((_NBLK * _KB, 2 * dim), jnp.float32),
    )(tt, tt, tt, tt)


def _sc_pool(xp, table_lin, seq, dim):
    batch, _ = xp.shape
    nw = _NCORE * _NSUB
    assert batch % nw == 0
    bpw = batch // nw
    assert bpw % 4 == 0
    wdim = dim // 2

    mesh = plsc.VectorSubcoreMesh(core_axis_name="c", subcore_axis_name="s")

    @functools.partial(
        pl.kernel,
        mesh=mesh,
        compiler_params=pltpu.CompilerParams(
            use_tc_tiling_on_sc=False, needs_layout_passes=False),
        out_type=jax.ShapeDtypeStruct((batch, dim), jnp.float32),
        scratch_types=[
            pltpu.VMEM((bpw, seq), jnp.int32),
            pltpu.VMEM((4, seq, wdim), jnp.float32),
            pltpu.VMEM((bpw, dim), jnp.float32),
            pltpu.SemaphoreType.DMA,
            pltpu.SemaphoreType.DMA,
            pltpu.SemaphoreType.DMA,
            pltpu.SemaphoreType.DMA,
        ],
    )
    def k(x_hbm, table_hbm, out_hbm, idx_v, rows_v, pooled_v,
          sem0, sem1, sem2, sem3):
        wid = lax.axis_index("s") * _NCORE + lax.axis_index("c")
        base = wid * bpw
        pltpu.sync_copy(x_hbm.at[pl.ds(base, bpw)], idx_v)

        sems = (sem0, sem1, sem2, sem3)

        def gather(r, buf):
            return pltpu.make_async_copy(
                table_hbm.at[idx_v.at[r]], rows_v.at[buf], sems[buf])

        def accum(r, buf):
            for u in range(wdim // _NLANE):
                acc = plsc.bitcast(
                    rows_v[buf, 0, pl.ds(u * _NLANE, _NLANE)], jnp.bfloat16)
                for s in range(1, seq):
                    acc = acc + plsc.bitcast(
                        rows_v[buf, s, pl.ds(u * _NLANE, _NLANE)], jnp.bfloat16)
                ea, eb = plsc.unpack(acc, format=plsc.PackFormat.INTERLEAVED)
                pooled_v[r, pl.ds(2 * u * _NLANE, _NLANE)] = ea * (1.0 / seq)
                pooled_v[r, pl.ds((2 * u + 1) * _NLANE, _NLANE)] = eb * (1.0 / seq)

        gather(0, 0).start()
        gather(1, 1).start()
        gather(2, 2).start()

        def outer(g, carry):
            r0 = 4 * g
            for bf in range(4):
                r = r0 + bf

                @pl.when(r + 3 < bpw)
                def _():
                    gather(r + 3, (bf + 3) % 4).start()

                gather(r, bf).wait()
                accum(r, bf)
            return carry

        lax.fori_loop(0, bpw // 4, outer, 0)
        pltpu.sync_copy(pooled_v, out_hbm.at[pl.ds(base, bpw)])

    return k(xp, table_lin)


def _tc_head(pooled, wt, b2):
    batch, dim = pooled.shape
    _, seq = wt.shape
    bb = 2048
    assert batch % bb == 0

    def body(p_ref, w_ref, b_ref, o_ref):
        logits = jnp.dot(p_ref[...], w_ref[...],
                         preferred_element_type=jnp.float32) + b_ref[...]
        o_ref[...] = jax.nn.sigmoid(logits)

    return pl.pallas_call(
        body,
        grid=(batch // bb,),
        in_specs=[
            pl.BlockSpec((bb, dim), lambda i: (i, 0)),
            pl.BlockSpec((dim, seq), lambda i: (0, 0)),
            pl.BlockSpec((1, seq), lambda i: (0, 0)),
        ],
        out_specs=pl.BlockSpec((bb, seq), lambda i: (i, 0)),
        out_shape=jax.ShapeDtypeStruct((batch, seq), jnp.float32),
    )(pooled, wt, b2)


_PERM = np.array(
    [16 * (c // 32) + c % 16 + 32 * ((c % 32) // 16)
     for c in range(64)], dtype=np.int32)


def kernel(x, table, W, b):
    vocab, dim = table.shape
    seq = x.shape[1]
    packed = _tc_pack(jnp.swapaxes(table, 0, 1))
    qq = packed.shape[0]
    table_lin = packed.reshape(4 * qq, dim // 2)
    xp = 4 * lax.rem(x, qq) + lax.div(x, qq)
    pooled = _sc_pool(xp, table_lin, seq, dim)
    wt_perm = jnp.take(W.T, jnp.asarray(_PERM), axis=0)
    return _tc_head(pooled, wt_perm, b.reshape(1, -1))
